# baseline (speedup 1.0000x reference)
import functools

import jax
import jax.numpy as jnp
from jax import lax
from jax.experimental import pallas as pl
from jax.experimental.pallas import tpu as pltpu
from jax.experimental.pallas import tpu_sc as plsc

_B = 16384
_K = 16
_ROWS_PER_LINE = 8
_NC = 2
_NS = 16
_NW = _NC * _NS
_BPW = _B // _NW
_CHUNK = 128
_NCHUNK = _BPW // _CHUNK
_L = 16


def _mf_body(w_hbm, h_hbm, ulines_hbm, ilines_hbm, uoffs_hbm, ioffs_hbm,
             out_hbm, ul_v, il_v, uo_v, io_v, ubuf, vbuf, out_v, sem):
    wid = lax.axis_index("s") * _NC + lax.axis_index("c")

    pltpu.sync_copy(ulines_hbm.at[wid], ul_v)
    pltpu.sync_copy(ilines_hbm.at[wid], il_v)
    pltpu.sync_copy(uoffs_hbm.at[wid], uo_v)
    pltpu.sync_copy(ioffs_hbm.at[wid], io_v)

    for j in range(_NCHUNK):
        cu = pltpu.async_copy(w_hbm.at[ul_v.at[j]], ubuf, sem)
        cv = pltpu.async_copy(h_hbm.at[il_v.at[j]], vbuf, sem)
        cu.wait()
        cv.wait()

        def _dot16(c, carry, j=j):
            rows = c * _L + lax.iota(jnp.int32, _L)
            ucol = uo_v[j, pl.ds(c * _L, _L)]
            icol = io_v[j, pl.ds(c * _L, _L)]
            acc = jnp.zeros((_L,), jnp.float32)
            for k in range(_K):
                u = plsc.load_gather(ubuf, [rows, ucol + k])
                v = plsc.load_gather(vbuf, [rows, icol + k])
                acc = acc + u * v
            out_v[pl.ds(j * _CHUNK + c * _L, _L)] = 1.0 / (1.0 + jnp.exp(-acc))
            return carry

        lax.fori_loop(0, _CHUNK // _L, _dot16, 0)

    pltpu.sync_copy(out_v, out_hbm.at[pl.ds(wid * _BPW, _BPW)])


_mf_call = functools.partial(
    pl.kernel,
    out_type=jax.ShapeDtypeStruct((_B,), jnp.float32),
    mesh=plsc.VectorSubcoreMesh(core_axis_name="c", subcore_axis_name="s"),
    scratch_types=[
        pltpu.VMEM((_NCHUNK, _CHUNK), jnp.int32),
        pltpu.VMEM((_NCHUNK, _CHUNK), jnp.int32),
        pltpu.VMEM((_NCHUNK, _CHUNK), jnp.int32),
        pltpu.VMEM((_NCHUNK, _CHUNK), jnp.int32),
        pltpu.VMEM((_CHUNK, 8 * _K), jnp.float32),
        pltpu.VMEM((_CHUNK, 8 * _K), jnp.float32),
        pltpu.VMEM((_BPW,), jnp.float32),
        pltpu.SemaphoreType.DMA,
    ],
    compiler_params=pltpu.CompilerParams(
        needs_layout_passes=False, use_tc_tiling_on_sc=True),
)(_mf_body)


def kernel(x, W, H):
    wl = W.reshape(-1, _ROWS_PER_LINE * _K)
    hl = H.reshape(-1, _ROWS_PER_LINE * _K)
    uidx = x[:, 0]
    iidx = x[:, 1]
    shape = (_NW, _NCHUNK, _CHUNK)
    ulines = (uidx // _ROWS_PER_LINE).reshape(shape)
    ilines = (iidx // _ROWS_PER_LINE).reshape(shape)
    uoffs = ((uidx % _ROWS_PER_LINE) * _K).reshape(shape)
    ioffs = ((iidx % _ROWS_PER_LINE) * _K).reshape(shape)
    return _mf_call(wl, hl, ulines, ilines, uoffs, ioffs)


# speedup vs baseline: 6.1853x; 6.1853x over previous
import functools

import jax
import jax.numpy as jnp
from jax import lax
from jax.experimental import pallas as pl
from jax.experimental.pallas import tpu as pltpu
from jax.experimental.pallas import tpu_sc as plsc

_B = 16384
_K = 16
_NC = 2
_NS = 16
_NW = _NC * _NS
_BPW = _B // _NW
_L = 16
_SEG = 128
_NWAVE = _BPW // _L


def _mf_body(wt_hbm, ht_hbm, uidx_hbm, iidx_hbm, out_hbm,
             uidx_v, iidx_v, ublk, vblk, out_v, sem):
    wid = lax.axis_index("s") * _NC + lax.axis_index("c")
    lane = lax.iota(jnp.int32, _L)

    pltpu.sync_copy(uidx_hbm.at[wid], uidx_v)
    pltpu.sync_copy(iidx_hbm.at[wid], iidx_v)

    def _wave(w, carry):
        row = w >> 3
        col0 = (w & 7) * _L
        uvec = uidx_v[row, pl.ds(col0, _L)]
        ivec = iidx_v[row, pl.ds(col0, _L)]
        copies = []
        for t in range(_L):
            us = jnp.sum(jnp.where(lane == t, uvec, 0))
            vs = jnp.sum(jnp.where(lane == t, ivec, 0))
            uoff = pl.multiple_of((us >> 7) * _SEG, _SEG)
            voff = pl.multiple_of((vs >> 7) * _SEG, _SEG)
            copies.append(pltpu.async_copy(
                wt_hbm.at[:, pl.ds(uoff, _SEG)], ublk.at[t], sem))
            copies.append(pltpu.async_copy(
                ht_hbm.at[:, pl.ds(voff, _SEG)], vblk.at[t], sem))
        for c in copies:
            c.wait()

        ucol = uvec & (_SEG - 1)
        icol = ivec & (_SEG - 1)
        acc = jnp.zeros((_L,), jnp.float32)
        for k in range(_K):
            kv = jnp.full((_L,), k, jnp.int32)
            u = plsc.load_gather(ublk, [lane, kv, ucol])
            v = plsc.load_gather(vblk, [lane, kv, icol])
            acc = acc + u * v
        out_v[pl.ds(w * _L, _L)] = 1.0 / (1.0 + jnp.exp(-acc))
        return carry

    lax.fori_loop(0, _NWAVE, _wave, 0)

    pltpu.sync_copy(out_v, out_hbm.at[pl.ds(wid * _BPW, _BPW)])


_mf_call = functools.partial(
    pl.kernel,
    out_type=jax.ShapeDtypeStruct((_B,), jnp.float32),
    mesh=plsc.VectorSubcoreMesh(core_axis_name="c", subcore_axis_name="s"),
    scratch_types=[
        pltpu.VMEM((_BPW // 128, 128), jnp.int32),
        pltpu.VMEM((_BPW // 128, 128), jnp.int32),
        pltpu.VMEM((_L, _K, _SEG), jnp.float32),
        pltpu.VMEM((_L, _K, _SEG), jnp.float32),
        pltpu.VMEM((_BPW,), jnp.float32),
        pltpu.SemaphoreType.DMA,
    ],
    compiler_params=pltpu.CompilerParams(
        needs_layout_passes=False, use_tc_tiling_on_sc=True),
)(_mf_body)


def kernel(x, W, H):
    wt = W.T
    ht = H.T
    shape = (_NW, _BPW // 128, 128)
    uidx = x[:, 0].reshape(shape)
    iidx = x[:, 1].reshape(shape)
    return _mf_call(wt, ht, uidx, iidx)


# speedup vs baseline: 6.1975x; 1.0020x over previous
import functools

import jax
import jax.numpy as jnp
from jax import lax
from jax.experimental import pallas as pl
from jax.experimental.pallas import tpu as pltpu
from jax.experimental.pallas import tpu_sc as plsc

_B = 16384
_K = 16
_NC = 2
_NS = 16
_NW = _NC * _NS
_BPW = _B // _NW
_L = 16
_SEG = 128
_WV = 8
_NWAVE = _BPW // _WV


def _mf_body(wt_hbm, ht_hbm, uidx_hbm, iidx_hbm, out_hbm,
             uidx_v, iidx_v, ublk, vblk, out_v, sem0, sem1):
    wid = lax.axis_index("s") * _NC + lax.axis_index("c")
    lane = lax.iota(jnp.int32, _L)
    sems = (sem0, sem1)

    pltpu.sync_copy(uidx_hbm.at[wid], uidx_v)
    pltpu.sync_copy(iidx_hbm.at[wid], iidx_v)

    def _load_idx(w):
        row = w >> 4
        col0 = (w & 15) * _WV
        uvec = uidx_v[row, pl.ds(col0, _L)]
        ivec = iidx_v[row, pl.ds(col0, _L)]
        return uvec, ivec

    def _fire(w, slot):
        uvec, ivec = _load_idx(w)
        for t in range(_WV):
            us = jnp.sum(jnp.where(lane == t, uvec, 0))
            vs = jnp.sum(jnp.where(lane == t, ivec, 0))
            uoff = pl.multiple_of((us >> 7) * _SEG, _SEG)
            voff = pl.multiple_of((vs >> 7) * _SEG, _SEG)
            pltpu.make_async_copy(
                wt_hbm.at[:, pl.ds(uoff, _SEG)], ublk.at[slot, t],
                sems[slot]).start()
            pltpu.make_async_copy(
                ht_hbm.at[:, pl.ds(voff, _SEG)], vblk.at[slot, t],
                sems[slot]).start()

    def _drain(slot):
        for t in range(_WV):
            pltpu.make_async_copy(
                wt_hbm.at[:, pl.ds(0, _SEG)], ublk.at[slot, t],
                sems[slot]).wait()
            pltpu.make_async_copy(
                ht_hbm.at[:, pl.ds(0, _SEG)], vblk.at[slot, t],
                sems[slot]).wait()

    def _compute(w, slot):
        uvec, ivec = _load_idx(w)
        ucol = uvec & (_SEG - 1)
        icol = ivec & (_SEG - 1)
        blk = lane & (_WV - 1)
        acc = jnp.zeros((_L,), jnp.float32)
        sv = jnp.full((_L,), slot, jnp.int32)
        for k in range(_K):
            kv = jnp.full((_L,), k, jnp.int32)
            u = plsc.load_gather(ublk, [sv, blk, kv, ucol])
            v = plsc.load_gather(vblk, [sv, blk, kv, icol])
            acc = acc + u * v
        res = 1.0 / (1.0 + jnp.exp(-acc))
        plsc.store_compressed(out_v.at[pl.ds(w * _WV, _L)], res, mask=lane < _WV)

    _fire(0, 0)

    def _wavepair(i, carry):
        w = i * 2
        _fire(w + 1, 1)
        _drain(0)
        _compute(w, 0)

        @pl.when(w + 2 < _NWAVE)
        def _():
            _fire(w + 2, 0)

        _drain(1)
        _compute(w + 1, 1)
        return carry

    lax.fori_loop(0, _NWAVE // 2, _wavepair, 0)

    pltpu.sync_copy(out_v.at[pl.ds(0, _BPW)], out_hbm.at[pl.ds(wid * _BPW, _BPW)])


_mf_call = functools.partial(
    pl.kernel,
    out_type=jax.ShapeDtypeStruct((_B,), jnp.float32),
    mesh=plsc.VectorSubcoreMesh(core_axis_name="c", subcore_axis_name="s"),
    scratch_types=[
        pltpu.VMEM((_BPW // 128, 128), jnp.int32),
        pltpu.VMEM((_BPW // 128, 128), jnp.int32),
        pltpu.VMEM((2, _WV, _K, _SEG), jnp.float32),
        pltpu.VMEM((2, _WV, _K, _SEG), jnp.float32),
        pltpu.VMEM((_BPW + _L, ), jnp.float32),
        pltpu.SemaphoreType.DMA,
        pltpu.SemaphoreType.DMA,
    ],
    compiler_params=pltpu.CompilerParams(
        needs_layout_passes=False, use_tc_tiling_on_sc=True),
)(_mf_body)


def kernel(x, W, H):
    wt = W.T
    ht = H.T
    shape = (_NW, _BPW // 128, 128)
    uidx = x[:, 0].reshape(shape)
    iidx = x[:, 1].reshape(shape)
    return _mf_call(wt, ht, uidx, iidx)
